# Initial kernel scaffold; baseline (speedup 1.0000x reference)
import jax
import jax.numpy as jnp
from jax.experimental import pallas as pl


def kernel(x, ei_seq, ei_knn, ei_dis, W0_seq, b0_seq, W0_knn, b0_knn, W0_dis, b0_dis, fc0_W, fc0_b, bn0_gamma, bn0_beta, W1_seq, b1_seq, W1_knn, b1_knn, W1_dis, b1_dis, fc1_W, fc1_b):
    raise NotImplementedError("write your pallas kernel here")


# speedup vs baseline: 7.9052x; 7.9052x over previous
import functools

import jax
import jax.numpy as jnp
from jax import lax
from jax.experimental import pallas as pl
from jax.experimental.pallas import tpu as pltpu
from jax.experimental.pallas import tpu_sc as plsc

N = 10000
D = 128
E = 320000
NT = 10240
NC, NS = 2, 16
NW = NC * NS
K = 128
WIN = 80
CH = 16
NCHUNK = WIN // CH
EP = NW * WIN * K
ROWS_PER_SUB = NT // NS
DEG_REGIONS = 8
DEG_FLAT = DEG_REGIONS * NT
DEG_PER_SUB = DEG_FLAT // NS
DWIN = 6 * EP // (NW * K)
BLK = 256
NBLK = NT // BLK
_PREC = lax.Precision.HIGHEST


def _mesh():
    return plsc.VectorSubcoreMesh(core_axis_name="c", subcore_axis_name="s")


def _sc_degrees(deg_idx):
    @functools.partial(
        pl.kernel,
        out_type=jax.ShapeDtypeStruct((NC, DEG_FLAT), jnp.float32),
        mesh=_mesh(),
        scratch_types=[
            pltpu.VMEM((DWIN, K), jnp.int32),
            pltpu.VMEM((K,), jnp.float32),
            pltpu.VMEM((K,), jnp.float32),
            pltpu.VMEM_SHARED((DEG_FLAT,), jnp.float32),
        ],
    )
    def k(idx_hbm, out_hbm, idx_v, ones_v, zeros_v, acc):
        c = lax.axis_index("c")
        s = lax.axis_index("s")
        wid = s * NC + c

        @pl.loop(0, K, step=16)
        def _(i):
            ones_v[pl.ds(i, 16)] = jnp.ones((16,), jnp.float32)
            zeros_v[pl.ds(i, 16)] = jnp.zeros((16,), jnp.float32)

        @pl.loop(0, DEG_PER_SUB, step=K)
        def _(i):
            pltpu.sync_copy(zeros_v, acc.at[pl.ds(s * DEG_PER_SUB + i, K)])

        plsc.subcore_barrier()
        pltpu.sync_copy(idx_hbm.at[wid], idx_v)

        @pl.loop(0, DWIN)
        def _(j):
            pltpu.sync_copy(ones_v, acc.at[idx_v.at[j]], add=True)

        plsc.subcore_barrier()
        pltpu.sync_copy(acc.at[pl.ds(s * DEG_PER_SUB, DEG_PER_SUB)],
                        out_hbm.at[c, pl.ds(s * DEG_PER_SUB, DEG_PER_SUB)])

    return k(deg_idx)


def _sc_scatter(table, srcp, dstp):
    @functools.partial(
        pl.kernel,
        out_type=jax.ShapeDtypeStruct((NC * 3 * NT, D), jnp.float32),
        mesh=_mesh(),
        scratch_types=[
            pltpu.VMEM((CH, K), jnp.int32),
            pltpu.VMEM((CH, K), jnp.int32),
            pltpu.VMEM((K, D), jnp.float32),
            pltpu.VMEM((K, D), jnp.float32),
            pltpu.VMEM_SHARED((NT, D), jnp.float32),
            pltpu.SemaphoreType.DMA,
            pltpu.SemaphoreType.DMA,
        ],
    )
    def k(table_hbm, srcp_hbm, dstp_hbm, out_hbm,
          src_idx, dst_idx, rows0, rows1, acc, sem0, sem1):
        c = lax.axis_index("c")
        s = lax.axis_index("s")
        wid = s * NC + c

        for r in range(3):
            @pl.loop(0, K)
            def _(i):
                @pl.loop(0, D, step=16)
                def _(j):
                    rows0[i, pl.ds(j, 16)] = jnp.zeros((16,), jnp.float32)

            @pl.loop(0, ROWS_PER_SUB, step=K)
            def _(i):
                pltpu.sync_copy(rows0, acc.at[pl.ds(s * ROWS_PER_SUB + i, K)])

            plsc.subcore_barrier()

            @pl.loop(0, NCHUNK)
            def _(ch):
                pltpu.sync_copy(
                    srcp_hbm.at[r * NW + wid, pl.ds(ch * CH, CH)], src_idx)
                pltpu.sync_copy(
                    dstp_hbm.at[r * NW + wid, pl.ds(ch * CH, CH)], dst_idx)

                pltpu.make_async_copy(
                    table_hbm.at[src_idx.at[0]], rows0, sem0).start()
                pltpu.make_async_copy(
                    table_hbm.at[src_idx.at[1]], rows1, sem1).start()

                @pl.loop(0, CH, step=2)
                def _(j):
                    pltpu.make_async_copy(
                        table_hbm.at[src_idx.at[j]], rows0, sem0).wait()
                    pltpu.sync_copy(rows0, acc.at[dst_idx.at[j]], add=True)

                    @pl.when(j + 2 < CH)
                    def _():
                        pltpu.make_async_copy(
                            table_hbm.at[src_idx.at[j + 2]], rows0,
                            sem0).start()

                    pltpu.make_async_copy(
                        table_hbm.at[src_idx.at[j + 1]], rows1, sem1).wait()
                    pltpu.sync_copy(rows1, acc.at[dst_idx.at[j + 1]],
                                    add=True)

                    @pl.when(j + 3 < CH)
                    def _():
                        pltpu.make_async_copy(
                            table_hbm.at[src_idx.at[j + 3]], rows1,
                            sem1).start()

            plsc.subcore_barrier()
            pltpu.sync_copy(
                acc.at[pl.ds(s * ROWS_PER_SUB, ROWS_PER_SUB)],
                out_hbm.at[pl.ds((c * 3 + r) * NT + s * ROWS_PER_SUB,
                                 ROWS_PER_SUB)])

    return k(table, srcp, dstp)


def _tc_prep(x_pad, degp):
    def body(x_ref, d_ref, t_ref, rf_ref):
        deg = d_ref[0] + d_ref[1]
        rf = lax.rsqrt(jnp.maximum(deg, 1.0))
        rf_ref[...] = rf
        x = x_ref[...]
        t_ref[...] = jnp.stack([x * rf[r][:, None] for r in range(3)])

    return pl.pallas_call(
        body,
        grid=(NBLK,),
        in_specs=[
            pl.BlockSpec((BLK, D), lambda i: (i, 0)),
            pl.BlockSpec((NC, DEG_REGIONS, BLK), lambda i: (0, 0, i)),
        ],
        out_specs=[
            pl.BlockSpec((3, BLK, D), lambda i: (0, i, 0)),
            pl.BlockSpec((DEG_REGIONS, BLK), lambda i: (0, i)),
        ],
        out_shape=[
            jax.ShapeDtypeStruct((3, NT, D), jnp.float32),
            jax.ShapeDtypeStruct((DEG_REGIONS, NT), jnp.float32),
        ],
    )(x_pad, degp)


def _tc_mid_a(p, rfac, W0, prm0, fc0_W):
    def body(p_ref, rf_ref, w_ref, prm_ref, fc_ref, h_ref, st_ref):
        i = pl.program_id(0)
        p_ = p_ref[...]
        rf = rf_ref[...]
        acc = jnp.zeros((BLK, D), jnp.float32)
        for r in range(3):
            agg = (p_[0, r] + p_[1, r]) * rf[3 + r][:, None]
            acc += jnp.dot(agg, w_ref[r], precision=_PREC,
                           preferred_element_type=jnp.float32)
        acc += (prm_ref[0] + prm_ref[1] + prm_ref[2])[None]
        h2 = jnp.dot(acc, fc_ref[...], precision=_PREC,
                     preferred_element_type=jnp.float32) + prm_ref[3][None]
        hr = jnp.maximum(h2, 0.0)
        rowid = i * BLK + lax.broadcasted_iota(jnp.int32, (BLK, 1), 0)
        hr = jnp.where(rowid < N, hr, 0.0)
        h_ref[...] = hr

        @pl.when(i == 0)
        def _():
            st_ref[...] = jnp.zeros((8, D), jnp.float32)

        st_ref[...] += jnp.concatenate(
            [jnp.sum(hr, axis=0)[None], jnp.sum(hr * hr, axis=0)[None],
             jnp.zeros((6, D), jnp.float32)], axis=0)

    return pl.pallas_call(
        body,
        grid=(NBLK,),
        in_specs=[
            pl.BlockSpec((NC, 3, BLK, D), lambda i: (0, 0, i, 0)),
            pl.BlockSpec((DEG_REGIONS, BLK), lambda i: (0, i)),
            pl.BlockSpec((3, D, D), lambda i: (0, 0, 0)),
            pl.BlockSpec((8, D), lambda i: (0, 0)),
            pl.BlockSpec((D, D), lambda i: (0, 0)),
        ],
        out_specs=[
            pl.BlockSpec((BLK, D), lambda i: (i, 0)),
            pl.BlockSpec((8, D), lambda i: (0, 0)),
        ],
        out_shape=[
            jax.ShapeDtypeStruct((NT, D), jnp.float32),
            jax.ShapeDtypeStruct((8, D), jnp.float32),
        ],
    )(p, rfac, W0, prm0, fc0_W)


def _tc_mid_b(hrelu, stats, prm0, rfac):
    def body(h_ref, st_ref, prm_ref, rf_ref, t_ref):
        i = pl.program_id(0)
        mu = st_ref[0] / N
        var = st_ref[1] / N - mu * mu
        sc = lax.rsqrt(var + 1e-5) * prm_ref[4]
        hb = (h_ref[...] - mu[None]) * sc[None] + prm_ref[5][None]
        rowid = i * BLK + lax.broadcasted_iota(jnp.int32, (BLK, 1), 0)
        hb = jnp.where(rowid < N, hb, 0.0)
        rf = rf_ref[...]
        t_ref[...] = jnp.stack([hb * rf[r][:, None] for r in range(3)])

    return pl.pallas_call(
        body,
        grid=(NBLK,),
        in_specs=[
            pl.BlockSpec((BLK, D), lambda i: (i, 0)),
            pl.BlockSpec((8, D), lambda i: (0, 0)),
            pl.BlockSpec((8, D), lambda i: (0, 0)),
            pl.BlockSpec((DEG_REGIONS, BLK), lambda i: (0, i)),
        ],
        out_specs=pl.BlockSpec((3, BLK, D), lambda i: (0, i, 0)),
        out_shape=jax.ShapeDtypeStruct((3, NT, D), jnp.float32),
    )(hrelu, stats, prm0, rfac)


def _tc_final(p, rfac, W1, prm1, fc1_W):
    def body(p_ref, rf_ref, w_ref, prm_ref, fc_ref, y_ref):
        p_ = p_ref[...]
        rf = rf_ref[...]
        acc = jnp.zeros((BLK, D), jnp.float32)
        for r in range(3):
            agg = (p_[0, r] + p_[1, r]) * rf[3 + r][:, None]
            acc += jnp.dot(agg, w_ref[r], precision=_PREC,
                           preferred_element_type=jnp.float32)
        acc += (prm_ref[0] + prm_ref[1] + prm_ref[2])[None]
        y_ref[...] = jnp.dot(acc, fc_ref[...], precision=_PREC,
                             preferred_element_type=jnp.float32) \
            + prm_ref[3][None]

    return pl.pallas_call(
        body,
        grid=(NBLK,),
        in_specs=[
            pl.BlockSpec((NC, 3, BLK, D), lambda i: (0, 0, i, 0)),
            pl.BlockSpec((DEG_REGIONS, BLK), lambda i: (0, i)),
            pl.BlockSpec((3, D, D), lambda i: (0, 0, 0)),
            pl.BlockSpec((8, D), lambda i: (0, 0)),
            pl.BlockSpec((D, D), lambda i: (0, 0)),
        ],
        out_specs=pl.BlockSpec((BLK, D), lambda i: (i, 0)),
        out_shape=jax.ShapeDtypeStruct((NT, D), jnp.float32),
    )(p, rfac, W1, prm1, fc1_W)


def _pad_edges(idx, region_offset):
    fill = (N + (jnp.arange(EP - E, dtype=jnp.int32) % (NT - N))
            + region_offset)
    return jnp.concatenate([idx + region_offset, fill]).reshape(NW, WIN, K)


def kernel(x, ei_seq, ei_knn, ei_dis,
           W0_seq, b0_seq, W0_knn, b0_knn, W0_dis, b0_dis,
           fc0_W, fc0_b, bn0_gamma, bn0_beta,
           W1_seq, b1_seq, W1_knn, b1_knn, W1_dis, b1_dis,
           fc1_W, fc1_b):
    eis = (ei_seq, ei_knn, ei_dis)

    deg_idx = jnp.concatenate(
        [_pad_edges(eis[r][side], (side * 3 + r) * NT).reshape(-1)
         for side in (0, 1) for r in range(3)]).reshape(NW, DWIN, K)

    srcp = jnp.concatenate([_pad_edges(eis[r][0], r * NT) for r in range(3)])
    dstp = jnp.concatenate([_pad_edges(eis[r][1], 0) for r in range(3)])

    degp = _sc_degrees(deg_idx).reshape(NC, DEG_REGIONS, NT)

    x_pad = jnp.pad(x, ((0, NT - N), (0, 0)))
    tables0, rfac = _tc_prep(x_pad, degp)

    p0 = _sc_scatter(tables0.reshape(3 * NT, D), srcp, dstp)
    p0 = p0.reshape(NC, 3, NT, D)

    W0 = jnp.stack([W0_seq, W0_knn, W0_dis])
    prm0 = jnp.stack([b0_seq, b0_knn, b0_dis, fc0_b, bn0_gamma, bn0_beta,
                      jnp.zeros_like(fc0_b), jnp.zeros_like(fc0_b)])
    hrelu, stats = _tc_mid_a(p0, rfac, W0, prm0, fc0_W)
    tables1 = _tc_mid_b(hrelu, stats, prm0, rfac)

    p1 = _sc_scatter(tables1.reshape(3 * NT, D), srcp, dstp)
    p1 = p1.reshape(NC, 3, NT, D)

    W1 = jnp.stack([W1_seq, W1_knn, W1_dis])
    prm1 = jnp.stack([b1_seq, b1_knn, b1_dis, fc1_b,
                      jnp.zeros_like(fc1_b), jnp.zeros_like(fc1_b),
                      jnp.zeros_like(fc1_b), jnp.zeros_like(fc1_b)])
    y = _tc_final(p1, rfac, W1, prm1, fc1_W)
    return y[:N]


# speedup vs baseline: 8.2131x; 1.0389x over previous
import functools

import jax
import jax.numpy as jnp
from jax import lax
from jax.experimental import pallas as pl
from jax.experimental.pallas import tpu as pltpu
from jax.experimental.pallas import tpu_sc as plsc

N = 10000
D = 128
E = 320000
NT = 10240
NC, NS = 2, 16
NW = NC * NS
K = 128
WIN = 80
CH = 40
NCHUNK = WIN // CH
EP = NW * WIN * K
ROWS_PER_SUB = NT // NS
DEG_REGIONS = 8
DEG_FLAT = DEG_REGIONS * NT
DEG_PER_SUB = DEG_FLAT // NS
DWIN = 6 * EP // (NW * K)
BLK = 256
NBLK = NT // BLK
_PREC = lax.Precision.HIGHEST


def _mesh():
    return plsc.VectorSubcoreMesh(core_axis_name="c", subcore_axis_name="s")


def _sc_degrees(deg_idx):
    @functools.partial(
        pl.kernel,
        out_type=jax.ShapeDtypeStruct((NC, DEG_FLAT), jnp.float32),
        mesh=_mesh(),
        scratch_types=[
            pltpu.VMEM((DWIN, K), jnp.int32),
            pltpu.VMEM((K,), jnp.float32),
            pltpu.VMEM((K,), jnp.float32),
            pltpu.VMEM_SHARED((DEG_FLAT,), jnp.float32),
        ],
    )
    def k(idx_hbm, out_hbm, idx_v, ones_v, zeros_v, acc):
        c = lax.axis_index("c")
        s = lax.axis_index("s")
        wid = s * NC + c

        @pl.loop(0, K, step=16)
        def _(i):
            ones_v[pl.ds(i, 16)] = jnp.ones((16,), jnp.float32)
            zeros_v[pl.ds(i, 16)] = jnp.zeros((16,), jnp.float32)

        @pl.loop(0, DEG_PER_SUB, step=K)
        def _(i):
            pltpu.sync_copy(zeros_v, acc.at[pl.ds(s * DEG_PER_SUB + i, K)])

        plsc.subcore_barrier()
        pltpu.sync_copy(idx_hbm.at[wid], idx_v)

        @pl.loop(0, DWIN)
        def _(j):
            pltpu.sync_copy(ones_v, acc.at[idx_v.at[j]], add=True)

        plsc.subcore_barrier()
        pltpu.sync_copy(acc.at[pl.ds(s * DEG_PER_SUB, DEG_PER_SUB)],
                        out_hbm.at[c, pl.ds(s * DEG_PER_SUB, DEG_PER_SUB)])

    return k(deg_idx)


def _sc_scatter(table, srcp, dstp):
    @functools.partial(
        pl.kernel,
        out_type=jax.ShapeDtypeStruct((NC * 3 * NT, D), jnp.float32),
        mesh=_mesh(),
        scratch_types=[
            pltpu.VMEM((CH, K), jnp.int32),
            pltpu.VMEM((CH, K), jnp.int32),
            pltpu.VMEM((K, D), jnp.float32),
            pltpu.VMEM((K, D), jnp.float32),
            pltpu.VMEM_SHARED((NT, D), jnp.float32),
            pltpu.SemaphoreType.DMA,
            pltpu.SemaphoreType.DMA,
            pltpu.SemaphoreType.DMA,
            pltpu.SemaphoreType.DMA,
        ],
    )
    def k(table_hbm, srcp_hbm, dstp_hbm, out_hbm,
          src_idx, dst_idx, rows0, rows1, acc, sem0, sem1, ssem0, ssem1):
        c = lax.axis_index("c")
        s = lax.axis_index("s")
        wid = s * NC + c

        for r in range(3):
            @pl.loop(0, K)
            def _(i):
                @pl.loop(0, D, step=16)
                def _(j):
                    rows0[i, pl.ds(j, 16)] = jnp.zeros((16,), jnp.float32)

            @pl.loop(0, ROWS_PER_SUB, step=K)
            def _(i):
                pltpu.sync_copy(rows0, acc.at[pl.ds(s * ROWS_PER_SUB + i, K)])

            plsc.subcore_barrier()

            @pl.loop(0, NCHUNK)
            def _(ch):
                pltpu.sync_copy(
                    srcp_hbm.at[r * NW + wid, pl.ds(ch * CH, CH)], src_idx)
                pltpu.sync_copy(
                    dstp_hbm.at[r * NW + wid, pl.ds(ch * CH, CH)], dst_idx)

                pltpu.make_async_copy(
                    table_hbm.at[src_idx.at[0]], rows0, sem0).start()
                pltpu.make_async_copy(
                    table_hbm.at[src_idx.at[1]], rows1, sem1).start()

                @pl.loop(0, CH, step=2)
                def _(j):
                    pltpu.make_async_copy(
                        table_hbm.at[src_idx.at[j]], rows0, sem0).wait()
                    pltpu.make_async_copy(
                        rows0, acc.at[dst_idx.at[j]], ssem0).start(add=True)
                    pltpu.make_async_copy(
                        table_hbm.at[src_idx.at[j + 1]], rows1, sem1).wait()
                    pltpu.make_async_copy(
                        rows1, acc.at[dst_idx.at[j + 1]], ssem1).start(add=True)

                    @pl.when(j + 2 < CH)
                    def _():
                        pltpu.make_async_copy(
                            rows0, acc.at[dst_idx.at[j]], ssem0).wait()
                        pltpu.make_async_copy(
                            table_hbm.at[src_idx.at[j + 2]], rows0,
                            sem0).start()

                    @pl.when(j + 3 < CH)
                    def _():
                        pltpu.make_async_copy(
                            rows1, acc.at[dst_idx.at[j + 1]], ssem1).wait()
                        pltpu.make_async_copy(
                            table_hbm.at[src_idx.at[j + 3]], rows1,
                            sem1).start()

                pltpu.make_async_copy(
                    rows0, acc.at[dst_idx.at[CH - 2]], ssem0).wait()
                pltpu.make_async_copy(
                    rows1, acc.at[dst_idx.at[CH - 1]], ssem1).wait()

            plsc.subcore_barrier()
            pltpu.sync_copy(
                acc.at[pl.ds(s * ROWS_PER_SUB, ROWS_PER_SUB)],
                out_hbm.at[pl.ds((c * 3 + r) * NT + s * ROWS_PER_SUB,
                                 ROWS_PER_SUB)])

    return k(table, srcp, dstp)


def _tc_prep(x_pad, degp):
    def body(x_ref, d_ref, t_ref, rf_ref):
        deg = d_ref[0] + d_ref[1]
        rf = lax.rsqrt(jnp.maximum(deg, 1.0))
        rf_ref[...] = rf
        x = x_ref[...]
        t_ref[...] = jnp.stack([x * rf[r][:, None] for r in range(3)])

    return pl.pallas_call(
        body,
        grid=(NBLK,),
        in_specs=[
            pl.BlockSpec((BLK, D), lambda i: (i, 0)),
            pl.BlockSpec((NC, DEG_REGIONS, BLK), lambda i: (0, 0, i)),
        ],
        out_specs=[
            pl.BlockSpec((3, BLK, D), lambda i: (0, i, 0)),
            pl.BlockSpec((DEG_REGIONS, BLK), lambda i: (0, i)),
        ],
        out_shape=[
            jax.ShapeDtypeStruct((3, NT, D), jnp.float32),
            jax.ShapeDtypeStruct((DEG_REGIONS, NT), jnp.float32),
        ],
    )(x_pad, degp)


def _tc_mid_a(p, rfac, W0, prm0, fc0_W):
    def body(p_ref, rf_ref, w_ref, prm_ref, fc_ref, h_ref, st_ref):
        i = pl.program_id(0)
        p_ = p_ref[...]
        rf = rf_ref[...]
        acc = jnp.zeros((BLK, D), jnp.float32)
        for r in range(3):
            agg = (p_[0, r] + p_[1, r]) * rf[3 + r][:, None]
            acc += jnp.dot(agg, w_ref[r], precision=_PREC,
                           preferred_element_type=jnp.float32)
        acc += (prm_ref[0] + prm_ref[1] + prm_ref[2])[None]
        h2 = jnp.dot(acc, fc_ref[...], precision=_PREC,
                     preferred_element_type=jnp.float32) + prm_ref[3][None]
        hr = jnp.maximum(h2, 0.0)
        rowid = i * BLK + lax.broadcasted_iota(jnp.int32, (BLK, 1), 0)
        hr = jnp.where(rowid < N, hr, 0.0)
        h_ref[...] = hr

        @pl.when(i == 0)
        def _():
            st_ref[...] = jnp.zeros((8, D), jnp.float32)

        st_ref[...] += jnp.concatenate(
            [jnp.sum(hr, axis=0)[None], jnp.sum(hr * hr, axis=0)[None],
             jnp.zeros((6, D), jnp.float32)], axis=0)

    return pl.pallas_call(
        body,
        grid=(NBLK,),
        in_specs=[
            pl.BlockSpec((NC, 3, BLK, D), lambda i: (0, 0, i, 0)),
            pl.BlockSpec((DEG_REGIONS, BLK), lambda i: (0, i)),
            pl.BlockSpec((3, D, D), lambda i: (0, 0, 0)),
            pl.BlockSpec((8, D), lambda i: (0, 0)),
            pl.BlockSpec((D, D), lambda i: (0, 0)),
        ],
        out_specs=[
            pl.BlockSpec((BLK, D), lambda i: (i, 0)),
            pl.BlockSpec((8, D), lambda i: (0, 0)),
        ],
        out_shape=[
            jax.ShapeDtypeStruct((NT, D), jnp.float32),
            jax.ShapeDtypeStruct((8, D), jnp.float32),
        ],
    )(p, rfac, W0, prm0, fc0_W)


def _tc_mid_b(hrelu, stats, prm0, rfac):
    def body(h_ref, st_ref, prm_ref, rf_ref, t_ref):
        i = pl.program_id(0)
        mu = st_ref[0] / N
        var = st_ref[1] / N - mu * mu
        sc = lax.rsqrt(var + 1e-5) * prm_ref[4]
        hb = (h_ref[...] - mu[None]) * sc[None] + prm_ref[5][None]
        rowid = i * BLK + lax.broadcasted_iota(jnp.int32, (BLK, 1), 0)
        hb = jnp.where(rowid < N, hb, 0.0)
        rf = rf_ref[...]
        t_ref[...] = jnp.stack([hb * rf[r][:, None] for r in range(3)])

    return pl.pallas_call(
        body,
        grid=(NBLK,),
        in_specs=[
            pl.BlockSpec((BLK, D), lambda i: (i, 0)),
            pl.BlockSpec((8, D), lambda i: (0, 0)),
            pl.BlockSpec((8, D), lambda i: (0, 0)),
            pl.BlockSpec((DEG_REGIONS, BLK), lambda i: (0, i)),
        ],
        out_specs=pl.BlockSpec((3, BLK, D), lambda i: (0, i, 0)),
        out_shape=jax.ShapeDtypeStruct((3, NT, D), jnp.float32),
    )(hrelu, stats, prm0, rfac)


def _tc_final(p, rfac, W1, prm1, fc1_W):
    def body(p_ref, rf_ref, w_ref, prm_ref, fc_ref, y_ref):
        p_ = p_ref[...]
        rf = rf_ref[...]
        acc = jnp.zeros((BLK, D), jnp.float32)
        for r in range(3):
            agg = (p_[0, r] + p_[1, r]) * rf[3 + r][:, None]
            acc += jnp.dot(agg, w_ref[r], precision=_PREC,
                           preferred_element_type=jnp.float32)
        acc += (prm_ref[0] + prm_ref[1] + prm_ref[2])[None]
        y_ref[...] = jnp.dot(acc, fc_ref[...], precision=_PREC,
                             preferred_element_type=jnp.float32) \
            + prm_ref[3][None]

    return pl.pallas_call(
        body,
        grid=(NBLK,),
        in_specs=[
            pl.BlockSpec((NC, 3, BLK, D), lambda i: (0, 0, i, 0)),
            pl.BlockSpec((DEG_REGIONS, BLK), lambda i: (0, i)),
            pl.BlockSpec((3, D, D), lambda i: (0, 0, 0)),
            pl.BlockSpec((8, D), lambda i: (0, 0)),
            pl.BlockSpec((D, D), lambda i: (0, 0)),
        ],
        out_specs=pl.BlockSpec((BLK, D), lambda i: (i, 0)),
        out_shape=jax.ShapeDtypeStruct((NT, D), jnp.float32),
    )(p, rfac, W1, prm1, fc1_W)


def _pad_edges(idx, region_offset):
    fill = (N + (jnp.arange(EP - E, dtype=jnp.int32) % (NT - N))
            + region_offset)
    return jnp.concatenate([idx + region_offset, fill]).reshape(NW, WIN, K)


def kernel(x, ei_seq, ei_knn, ei_dis,
           W0_seq, b0_seq, W0_knn, b0_knn, W0_dis, b0_dis,
           fc0_W, fc0_b, bn0_gamma, bn0_beta,
           W1_seq, b1_seq, W1_knn, b1_knn, W1_dis, b1_dis,
           fc1_W, fc1_b):
    eis = (ei_seq, ei_knn, ei_dis)

    deg_idx = jnp.concatenate(
        [_pad_edges(eis[r][side], (side * 3 + r) * NT).reshape(-1)
         for side in (0, 1) for r in range(3)]).reshape(NW, DWIN, K)

    srcp = jnp.concatenate([_pad_edges(eis[r][0], r * NT) for r in range(3)])
    dstp = jnp.concatenate([_pad_edges(eis[r][1], 0) for r in range(3)])

    degp = _sc_degrees(deg_idx).reshape(NC, DEG_REGIONS, NT)

    x_pad = jnp.pad(x, ((0, NT - N), (0, 0)))
    tables0, rfac = _tc_prep(x_pad, degp)

    p0 = _sc_scatter(tables0.reshape(3 * NT, D), srcp, dstp)
    p0 = p0.reshape(NC, 3, NT, D)

    W0 = jnp.stack([W0_seq, W0_knn, W0_dis])
    prm0 = jnp.stack([b0_seq, b0_knn, b0_dis, fc0_b, bn0_gamma, bn0_beta,
                      jnp.zeros_like(fc0_b), jnp.zeros_like(fc0_b)])
    hrelu, stats = _tc_mid_a(p0, rfac, W0, prm0, fc0_W)
    tables1 = _tc_mid_b(hrelu, stats, prm0, rfac)

    p1 = _sc_scatter(tables1.reshape(3 * NT, D), srcp, dstp)
    p1 = p1.reshape(NC, 3, NT, D)

    W1 = jnp.stack([W1_seq, W1_knn, W1_dis])
    prm1 = jnp.stack([b1_seq, b1_knn, b1_dis, fc1_b,
                      jnp.zeros_like(fc1_b), jnp.zeros_like(fc1_b),
                      jnp.zeros_like(fc1_b), jnp.zeros_like(fc1_b)])
    y = _tc_final(p1, rfac, W1, prm1, fc1_W)
    return y[:N]
